# baseline (speedup 1.0000x reference)
import functools

import jax
import jax.numpy as jnp
from jax import lax
from jax.experimental import pallas as pl
from jax.experimental.pallas import tpu as pltpu
from jax.experimental.pallas import tpu_sc as plsc

EMB_DIM = 64
NUM_WORKERS = 32
NB = 16


def _make_gather(n_batch: int, hist: int):
    b_per_w = n_batch // NUM_WORKERS
    n_chunks = b_per_w // NB
    mesh = plsc.VectorSubcoreMesh(core_axis_name="c", subcore_axis_name="s")

    @functools.partial(
        pl.kernel,
        mesh=mesh,
        out_type=jax.ShapeDtypeStruct((n_batch, hist, EMB_DIM), jnp.float32),
        scratch_types=[
            pltpu.VMEM((NB, hist), jnp.int32),
            pltpu.VMEM((NB, hist, EMB_DIM), jnp.float32),
            pltpu.SemaphoreType.DMA,
        ],
        compiler_params=pltpu.CompilerParams(use_tc_tiling_on_sc=False),
    )
    def gather_kernel(table_hbm, idx_hbm, out_hbm, idx_v, rows_v, sem):
        wid = lax.axis_index("s") * 2 + lax.axis_index("c")
        base = wid * b_per_w

        def body(c, carry):
            off = base + c * NB
            pltpu.sync_copy(idx_hbm.at[pl.ds(off, NB)], idx_v)
            for i in range(NB):
                pltpu.async_copy(table_hbm.at[idx_v.at[i]], rows_v.at[i], sem)
            for i in range(NB):
                pltpu.make_async_copy(
                    table_hbm.at[idx_v.at[i]], rows_v.at[i], sem
                ).wait()
            pltpu.sync_copy(rows_v, out_hbm.at[pl.ds(off, NB)])
            return carry

        lax.fori_loop(0, n_chunks, body, 0)

    return gather_kernel


def kernel(batch, table):
    b, h = batch.shape
    return _make_gather(b, h)(table, batch)
